# Initial kernel scaffold; baseline (speedup 1.0000x reference)
import jax
import jax.numpy as jnp
from jax.experimental import pallas as pl


def kernel(feat, edge_index, W, b):
    raise NotImplementedError("write your pallas kernel here")


# speedup vs baseline: 5.4237x; 5.4237x over previous
import functools

import jax
import jax.numpy as jnp
from jax import lax
from jax.experimental import pallas as pl
from jax.experimental.pallas import tpu as pltpu
from jax.experimental.pallas import tpu_sc as plsc

N_NODES = 10000
FEATS = 128
N_EDGES = 320000

NC = 2
NS = 16
NW = NC * NS
EPT = N_EDGES // NW
CHUNK = 128
NFULL = EPT // CHUNK
TAIL = EPT - NFULL * CHUNK
NPAD = 10112
ROWS_PT = NPAD // NS
DEGW = 128

_mesh = plsc.VectorSubcoreMesh(core_axis_name="c", subcore_axis_name="s",
                               num_cores=NC, num_subcores=NS)


def _deg_body(dst_hbm, ones_hbm, zeros_hbm, out_hbm, acc, idx_d, idx_dt, ones_v):
    cid = lax.axis_index("c")
    sid = lax.axis_index("s")
    ebase = (cid * NS + sid) * EPT
    rbase = sid * ROWS_PT

    pltpu.sync_copy(ones_hbm, ones_v)
    pltpu.sync_copy(zeros_hbm, acc.at[pl.ds(rbase, ROWS_PT)])
    plsc.subcore_barrier()

    @pl.loop(0, NFULL)
    def _chunk(j):
        off = ebase + j * CHUNK
        pltpu.sync_copy(dst_hbm.at[pl.ds(off, CHUNK)], idx_d)
        pltpu.sync_copy(ones_v, acc.at[idx_d], add=True)

    pltpu.sync_copy(dst_hbm.at[pl.ds(ebase + NFULL * CHUNK, TAIL)], idx_dt)
    pltpu.sync_copy(ones_v.at[pl.ds(0, TAIL)], acc.at[idx_dt], add=True)

    plsc.subcore_barrier()
    pltpu.sync_copy(
        acc.at[pl.ds(rbase, ROWS_PT)],
        out_hbm.at[pl.ds(cid * NPAD + rbase, ROWS_PT)],
    )


def _prop_body(x_hbm, src_hbm, dst_hbm, zeros_hbm, out_hbm,
                 acc, idx_s, idx_d, idx_st, idx_dt, rows, rows_t, sem):
    cid = lax.axis_index("c")
    sid = lax.axis_index("s")
    ebase = (cid * NS + sid) * EPT
    rbase = sid * ROWS_PT

    pltpu.sync_copy(zeros_hbm, acc.at[pl.ds(rbase, ROWS_PT)])
    plsc.subcore_barrier()

    @pl.loop(0, NFULL)
    def _chunk(j):
        off = ebase + j * CHUNK
        pltpu.sync_copy(src_hbm.at[pl.ds(off, CHUNK)], idx_s)
        pltpu.sync_copy(dst_hbm.at[pl.ds(off, CHUNK)], idx_d)
        pltpu.async_copy(x_hbm.at[idx_s], rows, sem).wait()
        pltpu.sync_copy(rows, acc.at[idx_d], add=True)

    toff = ebase + NFULL * CHUNK
    pltpu.sync_copy(src_hbm.at[pl.ds(toff, TAIL)], idx_st)
    pltpu.sync_copy(dst_hbm.at[pl.ds(toff, TAIL)], idx_dt)
    pltpu.async_copy(x_hbm.at[idx_st], rows_t, sem).wait()
    pltpu.sync_copy(rows_t, acc.at[idx_dt], add=True)

    plsc.subcore_barrier()
    pltpu.sync_copy(
        acc.at[pl.ds(rbase, ROWS_PT)],
        out_hbm.at[pl.ds(cid * NPAD + rbase, ROWS_PT)],
    )


_DEG_SCRATCH = [
    pltpu.VMEM_SHARED((NPAD, DEGW), jnp.float32),
    pltpu.VMEM((CHUNK,), jnp.int32),
    pltpu.VMEM((TAIL,), jnp.int32),
    pltpu.VMEM((CHUNK, DEGW), jnp.float32),
]
_PROP_SCRATCH = [
    pltpu.VMEM_SHARED((NPAD, FEATS), jnp.float32),
    pltpu.VMEM((CHUNK,), jnp.int32),
    pltpu.VMEM((CHUNK,), jnp.int32),
    pltpu.VMEM((TAIL,), jnp.int32),
    pltpu.VMEM((TAIL,), jnp.int32),
    pltpu.VMEM((CHUNK, FEATS), jnp.float32),
    pltpu.VMEM((TAIL, FEATS), jnp.float32),
    pltpu.SemaphoreType.DMA,
]

_deg_kernel = pl.kernel(
    _deg_body,
    out_type=jax.ShapeDtypeStruct((NC * NPAD, DEGW), jnp.float32),
    mesh=_mesh,
    scratch_types=_DEG_SCRATCH,
)

_prop_kernel = pl.kernel(
    _prop_body,
    out_type=jax.ShapeDtypeStruct((NC * NPAD, FEATS), jnp.float32),
    mesh=_mesh,
    scratch_types=_PROP_SCRATCH,
)


_RB = 1000


def _deg_of(dp0_ref, dp1_ref):
    return jnp.maximum(dp0_ref[:, 0:1] + dp1_ref[:, 0:1], 1.0)


def _scale0_body(dp0_ref, dp1_ref, f_ref, o_ref):
    o_ref[...] = f_ref[...] * lax.rsqrt(_deg_of(dp0_ref, dp1_ref))


def _scale_mid_body(dp0_ref, dp1_ref, r0_ref, r1_ref, o_ref):
    o_ref[...] = (r0_ref[...] + r1_ref[...]) / _deg_of(dp0_ref, dp1_ref)


def _final_body(dp0_ref, dp1_ref, r0_ref, r1_ref, w_ref, b_ref, o_ref):
    x = (r0_ref[...] + r1_ref[...]) * lax.rsqrt(_deg_of(dp0_ref, dp1_ref))
    o_ref[...] = lax.dot_general(
        x, w_ref[...], (((1,), (1,)), ((), ())),
        preferred_element_type=jnp.float32,
    ) + b_ref[...]


_row_spec = lambda w: pl.BlockSpec((_RB, w), lambda i: (i, 0))
_full_spec = lambda shape: pl.BlockSpec(shape, lambda i: (0,) * len(shape))

_scale0 = pl.pallas_call(
    _scale0_body,
    grid=(N_NODES // _RB,),
    in_specs=[_row_spec(DEGW), _row_spec(DEGW), _row_spec(FEATS)],
    out_specs=_row_spec(FEATS),
    out_shape=jax.ShapeDtypeStruct((N_NODES, FEATS), jnp.float32),
)

_scale_mid = pl.pallas_call(
    _scale_mid_body,
    grid=(N_NODES // _RB,),
    in_specs=[_row_spec(DEGW), _row_spec(DEGW), _row_spec(FEATS), _row_spec(FEATS)],
    out_specs=_row_spec(FEATS),
    out_shape=jax.ShapeDtypeStruct((N_NODES, FEATS), jnp.float32),
)

_final = pl.pallas_call(
    _final_body,
    grid=(N_NODES // _RB,),
    in_specs=[_row_spec(DEGW), _row_spec(DEGW), _row_spec(FEATS), _row_spec(FEATS),
              _full_spec((FEATS, FEATS)), _full_spec((1, FEATS))],
    out_specs=_row_spec(FEATS),
    out_shape=jax.ShapeDtypeStruct((N_NODES, FEATS), jnp.float32),
)


def kernel(feat, edge_index, W, b):
    src = edge_index[0].astype(jnp.int32)
    dst = edge_index[1].astype(jnp.int32)
    zeros_rows = jnp.zeros((ROWS_PT, FEATS), jnp.float32)
    zeros_deg = jnp.zeros((ROWS_PT, DEGW), jnp.float32)
    ones_rows = jnp.ones((CHUNK, DEGW), jnp.float32)

    degp = _deg_kernel(dst, ones_rows, zeros_deg)
    dp0, dp1 = degp[:N_NODES], degp[NPAD:NPAD + N_NODES]

    s0 = _scale0(dp0, dp1, feat)
    r1 = _prop_kernel(s0, src, dst, zeros_rows)
    s1 = _scale_mid(dp0, dp1, r1[:N_NODES], r1[NPAD:NPAD + N_NODES])
    r2 = _prop_kernel(s1, src, dst, zeros_rows)
    out = _final(dp0, dp1, r2[:N_NODES], r2[NPAD:NPAD + N_NODES], W,
                 b.reshape(1, FEATS))
    return out
